# Initial kernel scaffold; baseline (speedup 1.0000x reference)
import jax
import jax.numpy as jnp
from jax.experimental import pallas as pl


def kernel(index, value, field, emb_table, field_table, W, b):
    raise NotImplementedError("write your pallas kernel here")


# speedup vs baseline: 3.2416x; 3.2416x over previous
import jax
import jax.numpy as jnp
from jax import lax
from jax.experimental import pallas as pl
from jax.experimental.pallas import tpu as pltpu
from jax.experimental.pallas import tpu_sc as plsc

_B, _L = 4096, 100
_H = 64
_NC, _NS, _LN = 2, 16, 16
_NW = _NC * _NS
_RPW = _B // _NW
_EPW = _RPW * _L
_CH = 128
_NCH = _EPW // _CH
_GRP = _RPW // _LN

_BLK = 16384
_NVOC = 1000001
_NBLK = -(-_NVOC // _BLK)
_NPAD = _NBLK * _BLK


def _mv_body(tab_ref, ft_ref, w1_ref, w2_ref, e_ref, f_ref):
    y = jnp.dot(tab_ref[...], w1_ref[...], preferred_element_type=jnp.float32)
    e_ref[...] = y[:, 0]
    fy = jnp.dot(ft_ref[...], w2_ref[...], preferred_element_type=jnp.float32)
    f_ref[...] = fy[:, 0]


_mv_call = pl.pallas_call(
    _mv_body,
    grid=(_NBLK,),
    in_specs=[
        pl.BlockSpec((_BLK, _H), lambda i: (i, 0)),
        pl.BlockSpec((128, _H), lambda i: (0, 0)),
        pl.BlockSpec((_H, 1), lambda i: (0, 0)),
        pl.BlockSpec((_H, 1), lambda i: (0, 0)),
    ],
    out_specs=[
        pl.BlockSpec((_BLK,), lambda i: (i,)),
        pl.BlockSpec((128,), lambda i: (0,)),
    ],
    out_shape=[
        jax.ShapeDtypeStruct((_NPAD,), jnp.float32),
        jax.ShapeDtypeStruct((128,), jnp.float32),
    ],
)


def _sc_body(idx_hbm, fld_hbm, val_hbm, e_hbm, ftab_hbm, b_hbm, out_hbm,
             idx_v, fld_v, val_v, eg_v, ftab_v, b_v, out_v, gsem):
    wid = lax.axis_index("s") * _NC + lax.axis_index("c")
    base = wid * _EPW
    pltpu.sync_copy(idx_hbm.at[pl.ds(base, _EPW)], idx_v)

    def _fire(j, c):
        pltpu.make_async_copy(
            e_hbm.at[idx_v.at[pl.ds(j * _CH, _CH)]],
            eg_v.at[pl.ds(j * _CH, _CH)], gsem).start()
        return c

    lax.fori_loop(0, _NCH, _fire, 0)

    pltpu.sync_copy(fld_hbm.at[pl.ds(base, _EPW)], fld_v)
    pltpu.sync_copy(val_hbm.at[pl.ds(base, _EPW)], val_v)
    pltpu.sync_copy(ftab_hbm, ftab_v)
    pltpu.sync_copy(b_hbm, b_v)

    def _drain(j, c):
        pltpu.make_async_copy(
            e_hbm.at[idx_v.at[pl.ds(j * _CH, _CH)]],
            eg_v.at[pl.ds(j * _CH, _CH)], gsem).wait()
        return c

    lax.fori_loop(0, _NCH, _drain, 0)

    bias = b_v[...]
    lanes = lax.iota(jnp.int32, _LN) * _L
    for g in range(_GRP):
        def _acc(l, acc):
            pos = lanes + (g * _LN * _L + l)
            ev = plsc.load_gather(eg_v, [pos])
            fid = plsc.load_gather(fld_v, [pos])
            fv = plsc.load_gather(ftab_v, [fid])
            vv = plsc.load_gather(val_v, [pos])
            return acc + vv * (ev + fv)
        acc = lax.fori_loop(0, _L, _acc, jnp.zeros((_LN,), jnp.float32))
        out_v[pl.ds(g * _LN, _LN)] = acc + bias
    pltpu.sync_copy(out_v, out_hbm.at[pl.ds(wid * _RPW, _RPW)])


_sc_call = pl.kernel(
    _sc_body,
    out_type=jax.ShapeDtypeStruct((_B,), jnp.float32),
    mesh=plsc.VectorSubcoreMesh(core_axis_name="c", subcore_axis_name="s"),
    compiler_params=pltpu.CompilerParams(needs_layout_passes=False),
    scratch_types=[
        pltpu.VMEM((_EPW,), jnp.int32),
        pltpu.VMEM((_EPW,), jnp.int32),
        pltpu.VMEM((_EPW,), jnp.float32),
        pltpu.VMEM((_EPW,), jnp.float32),
        pltpu.VMEM((128,), jnp.float32),
        pltpu.VMEM((_LN,), jnp.float32),
        pltpu.VMEM((_RPW,), jnp.float32),
        pltpu.SemaphoreType.DMA,
    ],
)


def kernel(index, value, field, emb_table, field_table, W, b):
    idxf = index.reshape(-1).astype(jnp.int32)
    fldf = field.reshape(-1).astype(jnp.int32)
    valf = value.reshape(-1)
    w1 = W[:_H]
    w2 = W[_H:]
    ftpad = jnp.zeros((128, _H), jnp.float32).at[:field_table.shape[0]].set(field_table)
    e_tab, f_tab = _mv_call(emb_table, ftpad, w1, w2)
    b16 = jnp.broadcast_to(b, (_LN,))
    return _sc_call(idxf, fldf, valf, e_tab, f_tab, b16)


# speedup vs baseline: 22.4329x; 6.9203x over previous
import jax
import jax.numpy as jnp
from jax import lax
from jax.experimental import pallas as pl
from jax.experimental.pallas import tpu as pltpu
from jax.experimental.pallas import tpu_sc as plsc

_B, _L = 4096, 100
_H = 64
_NC, _NS, _LN = 2, 16, 16
_NW = _NC * _NS
_RPW = _B // _NW
_GRP = _RPW // _LN

_BLK = 32768
_NVOC = 1000001
_NBLK = -(-_NVOC // _BLK)
_NPAD = _NBLK * _BLK


def _mv_body(tabT_ref, ftT_ref, wT_ref, e_ref, f_ref):
    w = jnp.transpose(wT_ref[...])
    e_ref[...] = jnp.sum(tabT_ref[...] * w[:_H], axis=0)
    f_ref[...] = jnp.sum(ftT_ref[...] * w[_H:], axis=0)


_mv_call = pl.pallas_call(
    _mv_body,
    grid=(_NBLK,),
    in_specs=[
        pl.BlockSpec((_H, _BLK), lambda i: (0, i)),
        pl.BlockSpec((_H, 128), lambda i: (0, 0)),
        pl.BlockSpec((1, 128), lambda i: (0, 0)),
    ],
    out_specs=[
        pl.BlockSpec((_BLK,), lambda i: (i,)),
        pl.BlockSpec((128,), lambda i: (0,)),
    ],
    out_shape=[
        jax.ShapeDtypeStruct((_NPAD,), jnp.float32),
        jax.ShapeDtypeStruct((128,), jnp.float32),
    ],
)


def _sc_body(idx_hbm, fld_hbm, val_hbm, e_hbm, ftab_hbm, b_hbm, out_hbm,
             idx_v, fld_v, val_v, eg_v, ftab_v, b_v, out_v, gsem):
    wid = lax.axis_index("s") * _NC + lax.axis_index("c")
    c0 = wid * _RPW
    pltpu.sync_copy(idx_hbm.at[:, pl.ds(c0, _RPW)], idx_v)

    def _fire(l, c):
        pltpu.make_async_copy(e_hbm.at[idx_v.at[l]], eg_v.at[l], gsem).start()
        return c

    lax.fori_loop(0, _L, _fire, 0)
    pltpu.sync_copy(fld_hbm.at[:, pl.ds(c0, _RPW)], fld_v)
    pltpu.sync_copy(val_hbm.at[:, pl.ds(c0, _RPW)], val_v)
    pltpu.sync_copy(ftab_hbm, ftab_v)
    pltpu.sync_copy(b_hbm, b_v)

    bias = b_v[...]

    def _row(l, accs):
        pltpu.make_async_copy(e_hbm.at[idx_v.at[l]], eg_v.at[l], gsem).wait()
        out = []
        for g in range(_GRP):
            sl = pl.ds(g * _LN, _LN)
            ev = eg_v[l, sl]
            vv = val_v[l, sl]
            fid = fld_v[l, sl]
            fv = plsc.load_gather(ftab_v, [fid])
            out.append(accs[g] + vv * (ev + fv))
        return tuple(out)

    accs = lax.fori_loop(
        0, _L, _row, tuple(jnp.zeros((_LN,), jnp.float32) for _ in range(_GRP)))
    for g in range(_GRP):
        out_v[pl.ds(g * _LN, _LN)] = accs[g] + bias
    pltpu.sync_copy(out_v, out_hbm.at[pl.ds(wid * _RPW, _RPW)])


_sc_call = pl.kernel(
    _sc_body,
    out_type=jax.ShapeDtypeStruct((_B,), jnp.float32),
    mesh=plsc.VectorSubcoreMesh(core_axis_name="c", subcore_axis_name="s"),
    compiler_params=pltpu.CompilerParams(needs_layout_passes=False),
    scratch_types=[
        pltpu.VMEM((_L, _RPW), jnp.int32),
        pltpu.VMEM((_L, _RPW), jnp.int32),
        pltpu.VMEM((_L, _RPW), jnp.float32),
        pltpu.VMEM((_L, _RPW), jnp.float32),
        pltpu.VMEM((128,), jnp.float32),
        pltpu.VMEM((_LN,), jnp.float32),
        pltpu.VMEM((_RPW,), jnp.float32),
        pltpu.SemaphoreType.DMA,
    ],
)


def kernel(index, value, field, emb_table, field_table, W, b):
    e_tab, f_tab = _mv_call(emb_table.T, field_table.T, W.T)
    b16 = jnp.broadcast_to(b, (_LN,))
    return _sc_call(index.T, field.T, value.T, e_tab, f_tab, b16)
